# baseline (speedup 1.0000x reference)
import functools

import jax
import jax.numpy as jnp
from jax import lax
from jax.experimental import pallas as pl
from jax.experimental.pallas import tpu as pltpu
from jax.experimental.pallas import tpu_sc as plsc


def kernel(rel_pos_emb, seq_len):
    del seq_len
    V, D = rel_pos_emb.shape
    N = (V + 1) // 2

    info = plsc.get_sparse_core_info()
    NC, NS = info.num_cores, info.num_subcores
    NW = NC * NS
    rpw = N // 16
    JC = N // 2
    win = JC + 8 * (rpw - 1)

    mesh = plsc.VectorSubcoreMesh(core_axis_name="c", subcore_axis_name="s")

    @functools.partial(
        pl.kernel,
        mesh=mesh,
        out_type=jax.ShapeDtypeStruct((N, N, D), jnp.float32),
        scratch_types=[
            pltpu.VMEM((win, D), jnp.float32),
            pltpu.SemaphoreType.DMA,
        ],
    )
    def sliding_copy(t8_hbm, out_hbm, buf, sem):
        c = lax.axis_index("c")
        s = lax.axis_index("s")
        wid = s * NC + c
        rcls = wid % 8
        g = (wid // 8) % 2
        j0 = (wid // 16) * JC
        i_base = 7 - rcls + 8 * rpw * g
        b_last = N - 8 - 8 * (rpw * g + rpw - 1)
        base = pl.multiple_of(b_last + j0, 8)
        pltpu.sync_copy(t8_hbm.at[rcls, pl.ds(base, win)], buf)
        copies = []
        for m in range(rpw):
            off = 8 * (rpw - 1 - m)
            copies.append(
                pltpu.async_copy(
                    buf.at[pl.ds(off, JC)],
                    out_hbm.at[i_base + 8 * m, pl.ds(j0, JC)],
                    sem,
                )
            )
        for cp in copies:
            cp.wait()

    pad = jnp.concatenate(
        [rel_pos_emb, jnp.broadcast_to(rel_pos_emb[-1:], (2 * N + 7 - V, D))]
    )
    t8 = jnp.concatenate(
        [lax.slice_in_dim(pad, s, s + 2 * N) for s in range(8)], axis=0
    ).reshape(8, 2 * N, D)
    return sliding_copy(t8)


# speedup vs baseline: 1.0586x; 1.0586x over previous
import functools

import jax
import jax.numpy as jnp
from jax import lax
from jax.experimental import pallas as pl
from jax.experimental.pallas import tpu as pltpu
from jax.experimental.pallas import tpu_sc as plsc


def kernel(rel_pos_emb, seq_len):
    del seq_len
    V, D = rel_pos_emb.shape
    N = (V + 1) // 2

    info = plsc.get_sparse_core_info()
    NC, NS = info.num_cores, info.num_subcores
    NW = NC * NS
    rpw = N // 16
    JC = N // 2
    win = JC + 8 * (rpw - 1)

    mesh = plsc.VectorSubcoreMesh(core_axis_name="c", subcore_axis_name="s")

    @functools.partial(
        pl.kernel,
        mesh=mesh,
        out_type=jax.ShapeDtypeStruct((N, N, D), jnp.float32),
        scratch_types=[
            pltpu.VMEM((win, D), jnp.float32),
            pltpu.SemaphoreType.DMA,
        ],
    )
    def sliding_copy(t8_hbm, out_hbm, buf, sem):
        c = lax.axis_index("c")
        s = lax.axis_index("s")
        wid = s * NC + c
        rcls = wid % 8
        g = (wid // 8) % 2
        j0 = (wid // 16) * JC
        i_base = 7 - rcls + 8 * rpw * g
        b_last = N - 8 - 8 * (rpw * g + rpw - 1)
        base = pl.multiple_of(b_last + j0, 8)
        pltpu.sync_copy(t8_hbm.at[rcls, pl.ds(base, win)], buf)
        copies = []
        for m in range(rpw):
            off = 8 * (rpw - 1 - m)
            copies.append(
                pltpu.async_copy(
                    buf.at[pl.ds(off, JC)],
                    out_hbm.at[i_base + 8 * m, pl.ds(j0, JC)],
                    sem,
                )
            )
        for cp in copies:
            cp.wait()

    pad = jnp.concatenate(
        [rel_pos_emb, jnp.broadcast_to(rel_pos_emb[-1:], (2 * N - V, D))]
    )

    def t8_body(tab_ref, out_ref):
        x = tab_ref[...]
        for s in range(8):
            out_ref[s] = pltpu.roll(x, (2 * N - s) % (2 * N), 0)

    t8 = pl.pallas_call(
        t8_body,
        out_shape=jax.ShapeDtypeStruct((8, 2 * N, D), jnp.float32),
    )(pad)
    return sliding_copy(t8)


# speedup vs baseline: 1.0798x; 1.0200x over previous
import functools

import jax
import jax.numpy as jnp
from jax import lax
from jax.experimental import pallas as pl
from jax.experimental.pallas import tpu as pltpu
from jax.experimental.pallas import tpu_sc as plsc


def kernel(rel_pos_emb, seq_len):
    del seq_len
    V, D = rel_pos_emb.shape
    N = (V + 1) // 2

    info = plsc.get_sparse_core_info()
    NC, NS = info.num_cores, info.num_subcores
    NW = NC * NS
    rpw = N // 16
    JC = N // 2
    win = JC + 8 * (rpw - 1)

    mesh = plsc.VectorSubcoreMesh(core_axis_name="c", subcore_axis_name="s")

    @functools.partial(
        pl.kernel,
        mesh=mesh,
        out_type=jax.ShapeDtypeStruct((N, N, D), jnp.float32),
        scratch_types=[
            pltpu.VMEM((win, D), jnp.float32),
            pltpu.SemaphoreType.DMA,
        ],
    )
    def sliding_copy(t8_hbm, out_hbm, buf, sem):
        c = lax.axis_index("c")
        s = lax.axis_index("s")
        wid = s * NC + c
        rcls = wid % 8
        g = (wid // 8) % 2
        j0 = (wid // 16) * JC
        i_base = 7 - rcls + 8 * rpw * g
        b_last = N - 8 - 8 * (rpw * g + rpw - 1)
        base = pl.multiple_of(b_last + j0, 8)
        pltpu.sync_copy(t8_hbm.at[rcls, pl.ds(base, win)], buf)
        copies = []
        for m in range(rpw):
            off = 8 * (rpw - 1 - m)
            copies.append(
                pltpu.async_copy(
                    buf.at[pl.ds(off, JC)],
                    out_hbm.at[i_base + 8 * m, pl.ds(j0, JC)],
                    sem,
                )
            )
        for cp in copies:
            cp.wait()

    def t8_body(tab_ref, out_ref):
        x = tab_ref[...]
        xp = jnp.concatenate([x, x[V - 1 :]], axis=0)
        for s in range(8):
            out_ref[s] = pltpu.roll(xp, (2 * N - s) % (2 * N), 0)

    t8 = pl.pallas_call(
        t8_body,
        out_shape=jax.ShapeDtypeStruct((8, 2 * N, D), jnp.float32),
    )(rel_pos_emb)
    return sliding_copy(t8)
